# baseline (speedup 1.0000x reference)
import jax
import jax.numpy as jnp
from jax.experimental import pallas as pl

_BLK = 1536
_R0 = 0
_R1 = 16385


def _copy_scatter_body(x_ref, o_ref):
    i = pl.program_id(0)
    o_ref[...] = x_ref[...]

    @pl.when(i == _R0 // _BLK)
    def _():
        r = _R0 % _BLK
        o_ref[pl.ds(r, 1), :] = o_ref[pl.ds(r, 1), :] + jnp.float32(1.0)

    @pl.when(i == _R1 // _BLK)
    def _():
        r = _R1 % _BLK
        o_ref[pl.ds(r, 1), :] = o_ref[pl.ds(r, 1), :] + jnp.float32(1.0)


def kernel(x):
    n, s, d = x.shape
    y = jnp.transpose(x, (1, 0, 2)).reshape(s * n, d)
    out = pl.pallas_call(
        _copy_scatter_body,
        out_shape=jax.ShapeDtypeStruct((s * n, d), x.dtype),
        grid=(s * n // _BLK,),
        in_specs=[pl.BlockSpec((_BLK, d), lambda i: (i, 0))],
        out_specs=pl.BlockSpec((_BLK, d), lambda i: (i, 0)),
    )(y)
    return jnp.transpose(out.reshape(s, n, d), (1, 0, 2))


# speedup vs baseline: 1.0090x; 1.0090x over previous
import jax
import jax.numpy as jnp
from jax.experimental import pallas as pl
from jax.experimental.pallas import tpu as pltpu

_BLK = 3072
_R0 = 0
_R1 = 16385


def _copy_scatter_body(x_ref, o_ref):
    i = pl.program_id(0)
    o_ref[...] = x_ref[...]

    @pl.when(i == _R0 // _BLK)
    def _():
        r = _R0 % _BLK
        o_ref[pl.ds(r, 1), :] = o_ref[pl.ds(r, 1), :] + jnp.float32(1.0)

    @pl.when(i == _R1 // _BLK)
    def _():
        r = _R1 % _BLK
        o_ref[pl.ds(r, 1), :] = o_ref[pl.ds(r, 1), :] + jnp.float32(1.0)


def kernel(x):
    n, s, d = x.shape
    y = jnp.transpose(x, (1, 0, 2)).reshape(s * n, d)
    out = pl.pallas_call(
        _copy_scatter_body,
        out_shape=jax.ShapeDtypeStruct((s * n, d), x.dtype),
        grid=(s * n // _BLK,),
        in_specs=[pl.BlockSpec((_BLK, d), lambda i: (i, 0))],
        out_specs=pl.BlockSpec((_BLK, d), lambda i: (i, 0)),
        compiler_params=pltpu.CompilerParams(dimension_semantics=("parallel",)),
    )(y)
    return jnp.transpose(out.reshape(s, n, d), (1, 0, 2))
